# baseline (speedup 1.0000x reference)
import jax
import jax.numpy as jnp
from jax.experimental import pallas as pl


_SEQ_BLOCK = 512


def _body(pos_ref, out_ref):
    blk = pos_ref[...]
    for b in range(out_ref.shape[0]):
        out_ref[b] = blk


def kernel(x, pos_table):
    batch, seq_len = x.shape
    d_model = pos_table.shape[1]
    pos = pos_table[:seq_len]
    blk = _SEQ_BLOCK if seq_len % _SEQ_BLOCK == 0 else seq_len
    return pl.pallas_call(
        _body,
        grid=(seq_len // blk,),
        in_specs=[pl.BlockSpec((blk, d_model), lambda i: (i, 0))],
        out_specs=pl.BlockSpec((batch, blk, d_model), lambda i: (0, i, 0)),
        out_shape=jax.ShapeDtypeStruct((batch, seq_len, d_model), pos_table.dtype),
    )(pos)


# speedup vs baseline: 1.1467x; 1.1467x over previous
import jax
import jax.numpy as jnp
from jax.experimental import pallas as pl


_SEQ_BLOCK = 2048


def _body(pos_ref, out_ref):
    blk = pos_ref[...]
    for b in range(out_ref.shape[0]):
        out_ref[b] = blk


def kernel(x, pos_table):
    batch, seq_len = x.shape
    d_model = pos_table.shape[1]
    pos = pos_table[:seq_len]
    blk = _SEQ_BLOCK if seq_len % _SEQ_BLOCK == 0 else seq_len
    return pl.pallas_call(
        _body,
        grid=(seq_len // blk,),
        in_specs=[pl.BlockSpec((blk, d_model), lambda i: (i, 0))],
        out_specs=pl.BlockSpec((batch, blk, d_model), lambda i: (0, i, 0)),
        out_shape=jax.ShapeDtypeStruct((batch, seq_len, d_model), pos_table.dtype),
    )(pos)
